# baseline (speedup 1.0000x reference)
import functools

import jax
import jax.numpy as jnp
import numpy as np
from jax import lax
from jax.experimental import pallas as pl
from jax.experimental.pallas import tpu as pltpu
from jax.experimental.pallas import tpu_sc as plsc

N = 4096
NSQ = N * N
BAND = 256
NBANDS = N // BAND
BLK = 1024
NBLK = NSQ // BLK
BPR = N // BLK
STEP = 2.5
MAXK = 1024

NCORE = 2
NSUB = 16
L = 16
BPT = NBLK // NSUB
CAPB = 160
CAPC = 256
SPC = 2048
SLICE = SPC // NSUB
IMIN = np.int32(-2147483648)
IMAX = np.int32(2147483647)


def _maxes_copy_kernel(g_ref, a_ref, m_ref, out_ref):
    b = pl.program_id(0)
    g = g_ref[...]
    rows = lax.broadcasted_iota(jnp.int32, (BAND, N), 0) + b * BAND
    cols = lax.broadcasted_iota(jnp.int32, (BAND, N), 1)
    s = jnp.where(cols > rows, jnp.abs(g), 0.0)
    m_ref[...] = jnp.max(s.reshape(BAND, BPR, BLK), axis=-1)
    out_ref[...] = jnp.where(cols == rows, jnp.float32(-10.0), a_ref[...])


def _thresh_kernel(topk_ref, m_ref, t_ref):
    keys = lax.bitcast_convert_type(m_ref[...], jnp.int32)
    target = jnp.minimum(topk_ref[0], jnp.int32(MAXK))

    def body(_, lohi):
        lo, hi = lohi
        mid = lo + (hi - lo) // 2
        c = jnp.sum((keys >= mid).astype(jnp.int32))
        ok = c >= target
        return jnp.where(ok, mid, lo), jnp.where(ok, hi, mid)

    lo, _ = lax.fori_loop(0, 31, body, (jnp.int32(0), IMAX))
    for i in range(L):
        t_ref[i] = jnp.where(i == 1, target, lo)


def _lane_iota():
    return lax.iota(jnp.int32, L)


def _sc_body(grad_ref, mx_ref, tk_ref, anew_ref, out_ref,
             mxv, idbuf, blk, lraw, lidx, keys, kidx, tkv, stat,
             avbuf, valbuf,
             sp_raw, sp_idx, cnt_smem, lcnt_smem, sem_blk, sem_edge):
    sid = lax.axis_index("s")
    cid = lax.axis_index("c")
    lanes = _lane_iota()
    zi = jnp.zeros((L,), jnp.int32)

    @pl.when(sid == 0)
    def _():
        cnt_smem[0] = 0
    stat[pl.ds(0, L)] = zi
    for q in range(SLICE // L):
        zoff = pl.multiple_of(sid * SLICE + q * L, 8)
        pltpu.sync_copy(stat.at[pl.ds(0, L)], sp_raw.at[pl.ds(zoff, L)])
        pltpu.sync_copy(stat.at[pl.ds(0, L)], sp_idx.at[pl.ds(zoff, L)])
    for q in range(CAPB // L):
        idbuf[pl.ds(q * L, L)] = zi
    plsc.subcore_barrier()

    pltpu.sync_copy(mx_ref.at[pl.ds(pl.multiple_of(sid * BPT, 8), BPT)], mxv)
    pltpu.sync_copy(tk_ref, tkv)
    tk16 = tkv[...]
    t_key = jnp.sum(jnp.where(lanes == 0, tk16, 0))
    k_target = jnp.sum(jnp.where(lanes == 1, tk16, 0))

    def scan_body(g, cnt):
        m16 = mxv[pl.ds(pl.multiple_of(g * L, 8), L)]
        kk = lax.bitcast_convert_type(m16, jnp.int32)
        m = kk >= t_key
        ids = sid * BPT + g * L + lanes
        mi = m.astype(jnp.int32)
        wpos = cnt + plsc.cumsum(mi) - mi
        plsc.store_scatter(idbuf, [jnp.minimum(wpos, CAPB - 1)], ids, mask=m)
        return cnt + jnp.sum(mi)

    cnt = lax.fori_loop(0, BPT // L, scan_body, jnp.int32(0))
    cnt = jnp.minimum(cnt, jnp.int32(CAPB))

    lcnt_smem[0] = 0

    def chunk_body(c, _):
        @pl.when(c * L < cnt)
        def _():
            bidv = idbuf[pl.ds(pl.multiple_of(c * L, 8), L)]
            pltpu.async_copy(grad_ref.at[bidv], blk, sem_blk).wait()
            for r in range(L):
                @pl.when(c * L + r < cnt)
                def _():
                    bid = jnp.sum(jnp.where(lanes == r, bidv, 0))

                    def grp_body(g, lc):
                        raw = blk[r, pl.ds(pl.multiple_of(g * L, 8), L)]
                        bits = lax.bitcast_convert_type(raw, jnp.int32)
                        key = bits & 0x7FFFFFFF
                        pos = bid * BLK + g * L + lanes
                        row = pos >> 12
                        col = pos & (N - 1)
                        sel = (col > row) & (key >= t_key)
                        si = sel.astype(jnp.int32)
                        wpos = jnp.minimum(lc + plsc.cumsum(si) - si,
                                           CAPC - 1)
                        plsc.store_scatter(lraw, [wpos], bits, mask=sel)
                        plsc.store_scatter(lidx, [wpos], pos, mask=sel)
                        return lc + jnp.sum(si)

                    lc = lax.fori_loop(0, BLK // L, grp_body, lcnt_smem[0])
                    lcnt_smem[0] = jnp.minimum(lc, jnp.int32(CAPC))
        return 0

    lax.fori_loop(0, CAPB // L, chunk_body, 0)
    lcnt = lcnt_smem[0]

    lpad = (lcnt + (L - 1)) & ~(L - 1)

    @pl.when(lpad > lcnt)
    def _():
        toff = pl.multiple_of(lpad - L, 8)
        tail = lraw[pl.ds(toff, L)]
        m = (lpad - L + lanes) < lcnt
        lraw[pl.ds(toff, L)] = jnp.where(m, tail, 0)
        tidx = lidx[pl.ds(toff, L)]
        lidx[pl.ds(toff, L)] = jnp.where(m, tidx, 0)

    off = plsc.fetch_and_add(cnt_smem.at[0], lpad, subcore_id=0)
    off = jnp.minimum(off, jnp.int32(SPC - CAPC))

    def pack_body(q, _):
        @pl.when(q * L < lpad)
        def _():
            poff = pl.multiple_of(off + q * L, 8)
            pltpu.sync_copy(lraw.at[pl.ds(q * L, L)],
                            sp_raw.at[pl.ds(poff, L)])
            pltpu.sync_copy(lidx.at[pl.ds(q * L, L)],
                            sp_idx.at[pl.ds(poff, L)])
        return 0

    lax.fori_loop(0, CAPC // L, pack_body, 0)
    plsc.subcore_barrier()

    pltpu.sync_copy(sp_raw, keys)

    def count_ge(x):
        def cbody(q, acc):
            kk = keys[pl.ds(pl.multiple_of(q * L, 8), L)] & 0x7FFFFFFF
            return acc + jnp.sum((kk >= x).astype(jnp.int32))
        return lax.fori_loop(0, SPC // L, cbody, jnp.int32(0))

    def bis_body(_, lohi):
        lo, hi = lohi
        mid = lo + (hi - lo) // 2
        ok = count_ge(mid) >= k_target
        return jnp.where(ok, mid, lo), jnp.where(ok, hi, mid)

    tfin, _ = lax.fori_loop(0, 31, bis_body, (t_key, IMAX))

    pltpu.sync_copy(sp_idx, kidx)
    base = sid * SLICE

    for gq in range(SLICE // L):
        goff = pl.multiple_of(base + gq * L, 8)
        raw = keys[pl.ds(goff, L)]
        idx = kidx[pl.ds(goff, L)]
        key = raw & 0x7FFFFFFF
        sel = (key >= tfin) & ((idx & 1) == cid)

        @pl.when(jnp.sum(sel.astype(jnp.int32)) > 0)
        def _(raw=raw, idx=idx, sel=sel):
            fl = plsc.all_reduce_ffs(sel)
            first_idx = jnp.sum(jnp.where(lanes == fl, idx, 0))
            raw_first = jnp.sum(jnp.where(lanes == fl, raw, 0))
            idx_s = jnp.where(sel, idx, first_idx)
            raw_s = jnp.where(sel, raw, raw_first)
            u = idx_s >> 12
            v = idx_s & (N - 1)
            idx_t = (v << 12) | u
            pltpu.async_copy(anew_ref.at[idx_s], avbuf, sem_edge).wait()
            av = avbuf[...]
            gv = lax.bitcast_convert_type(raw_s, jnp.float32)
            exist = av > 0.0
            dec = exist & (gv <= 0.0)
            inc = (~exist) & (gv >= 0.0)
            d = jnp.where(dec, jnp.float32(-STEP),
                          jnp.where(inc, jnp.float32(STEP), jnp.float32(0.0)))
            valbuf[...] = av + d
            pltpu.async_copy(valbuf, anew_ref.at[idx_t], sem_edge).wait()
            pltpu.async_copy(valbuf, anew_ref.at[idx_s], sem_edge).wait()

    @pl.when((sid == 0) & (cid == 0))
    def _():
        stat[pl.ds(0, L)] = jnp.full((L,), tfin, jnp.int32)
        pltpu.sync_copy(stat.at[pl.ds(0, L)], out_ref)


@jax.jit
def _impl(gradA, A_logits, topk):
    maxes, a_new = pl.pallas_call(
        _maxes_copy_kernel,
        grid=(NBANDS,),
        in_specs=[
            pl.BlockSpec((BAND, N), lambda b: (b, 0)),
            pl.BlockSpec((BAND, N), lambda b: (b, 0)),
        ],
        out_specs=[
            pl.BlockSpec((BAND, BPR), lambda b: (b, 0)),
            pl.BlockSpec((BAND, N), lambda b: (b, 0)),
        ],
        out_shape=[
            jax.ShapeDtypeStruct((N, BPR), jnp.float32),
            jax.ShapeDtypeStruct((N, N), jnp.float32),
        ],
    )(gradA, A_logits)

    topk_arr = jnp.asarray(topk, jnp.int32).reshape((1,))
    tk = pl.pallas_call(
        _thresh_kernel,
        in_specs=[
            pl.BlockSpec(memory_space=pltpu.SMEM),
            pl.BlockSpec((N, BPR), lambda: (0, 0)),
        ],
        out_specs=pl.BlockSpec(memory_space=pltpu.SMEM),
        out_shape=jax.ShapeDtypeStruct((L,), jnp.int32),
    )(topk_arr, maxes)

    sc = pl.kernel(
        _sc_body,
        out_type=jax.ShapeDtypeStruct((L,), jnp.int32),
        mesh=plsc.VectorSubcoreMesh(
            core_axis_name="c", subcore_axis_name="s",
            num_cores=NCORE, num_subcores=NSUB),
        compiler_params=pltpu.CompilerParams(needs_layout_passes=False),
        scratch_types=[
            pltpu.VMEM((BPT,), jnp.float32),
            pltpu.VMEM((CAPB,), jnp.int32),
            pltpu.VMEM((L, BLK), jnp.float32),
            pltpu.VMEM((CAPC,), jnp.int32),
            pltpu.VMEM((CAPC,), jnp.int32),
            pltpu.VMEM((SPC,), jnp.int32),
            pltpu.VMEM((SPC,), jnp.int32),
            pltpu.VMEM((L,), jnp.int32),
            pltpu.VMEM((L,), jnp.int32),
            pltpu.VMEM((L,), jnp.float32),
            pltpu.VMEM((L,), jnp.float32),
            pltpu.VMEM_SHARED((SPC,), jnp.int32),
            pltpu.VMEM_SHARED((SPC,), jnp.int32),
            pltpu.SMEM((1,), jnp.int32),
            pltpu.SMEM((1,), jnp.int32),
            pltpu.SemaphoreType.DMA,
            pltpu.SemaphoreType.DMA,
        ],
    )
    grad2d = gradA.reshape(NBLK, BLK)
    mx_flat = maxes.reshape(NBLK)
    anew_ref = jax.new_ref(a_new.reshape(NSQ))
    _ = sc(grad2d, mx_flat, tk, anew_ref)
    return anew_ref[...].reshape(N, N)


def kernel(gradA, A_logits, topk):
    return _impl(gradA, A_logits, topk)


# speedup vs baseline: 1.1481x; 1.1481x over previous
import functools

import jax
import jax.numpy as jnp
import numpy as np
from jax import lax
from jax.experimental import pallas as pl
from jax.experimental.pallas import tpu as pltpu
from jax.experimental.pallas import tpu_sc as plsc

N = 4096
NSQ = N * N
BAND = 256
NBANDS = N // BAND
BLK = 1024
NBLK = NSQ // BLK
BPR = N // BLK
STEP = 2.5
MAXK = 1024

NCORE = 2
NSUB = 16
L = 16
BPT = NBLK // NSUB
CAPB = 160
CAPC = 256
SPC = 2048
SLICE = SPC // NSUB
IMIN = np.int32(-2147483648)
IMAX = np.int32(2147483647)


def _maxes_copy_kernel(g_ref, a_ref, m_ref, out_ref):
    b = pl.program_id(0)
    g = g_ref[...]
    rows = lax.broadcasted_iota(jnp.int32, (BAND, N), 0) + b * BAND
    cols = lax.broadcasted_iota(jnp.int32, (BAND, N), 1)
    s = jnp.where(cols > rows, jnp.abs(g), 0.0)
    m = jnp.max(s.reshape(BAND, BPR, BLK), axis=-1)
    m_ref[...] = m.reshape(BAND * BPR // 128, 128)
    a = jnp.where(cols == rows, jnp.float32(-10.0), a_ref[...])
    out_ref[...] = a.reshape(BAND * N // 128, 128)


def _thresh_kernel(topk_ref, m_ref, t_ref):
    keys = lax.bitcast_convert_type(m_ref[...], jnp.int32)
    target = jnp.minimum(topk_ref[0], jnp.int32(MAXK))

    def body(_, lohi):
        lo, hi = lohi
        mid = lo + (hi - lo) // 2
        c = jnp.sum((keys >= mid).astype(jnp.int32))
        ok = c >= target
        return jnp.where(ok, mid, lo), jnp.where(ok, hi, mid)

    lo, _ = lax.fori_loop(0, 31, body, (jnp.int32(0), IMAX))
    for i in range(L):
        t_ref[i] = jnp.where(i == 1, target, lo)


def _lane_iota():
    return lax.iota(jnp.int32, L)


def _sc_body(grad_ref, mx_ref, tk_ref, anew_ref, out_ref,
             mxv, idbuf, blk, lraw, lidx, keys, kidx, tkv, stat,
             avbuf, valbuf,
             sp_raw, sp_idx, cnt_smem, lcnt_smem, sem_blk, sem_edge):
    sid = lax.axis_index("s")
    cid = lax.axis_index("c")
    lanes = _lane_iota()
    zi = jnp.zeros((L,), jnp.int32)

    @pl.when(sid == 0)
    def _():
        cnt_smem[0] = 0
    stat[pl.ds(0, L)] = zi
    for q in range(SLICE // L):
        zoff = pl.multiple_of(sid * SLICE + q * L, 8)
        pltpu.sync_copy(stat.at[pl.ds(0, L)], sp_raw.at[pl.ds(zoff, L)])
        pltpu.sync_copy(stat.at[pl.ds(0, L)], sp_idx.at[pl.ds(zoff, L)])
    for q in range(CAPB // L):
        idbuf[pl.ds(q * L, L)] = zi
    plsc.subcore_barrier()

    pltpu.sync_copy(mx_ref.at[pl.ds(pl.multiple_of(sid * BPT, 8), BPT)], mxv)
    pltpu.sync_copy(tk_ref, tkv)
    tk16 = tkv[...]
    t_key = jnp.sum(jnp.where(lanes == 0, tk16, 0))
    k_target = jnp.sum(jnp.where(lanes == 1, tk16, 0))

    def scan_body(g, cnt):
        m16 = mxv[pl.ds(pl.multiple_of(g * L, 8), L)]
        kk = lax.bitcast_convert_type(m16, jnp.int32)
        m = kk >= t_key
        ids = sid * BPT + g * L + lanes
        mi = m.astype(jnp.int32)
        wpos = cnt + plsc.cumsum(mi) - mi
        plsc.store_scatter(idbuf, [jnp.minimum(wpos, CAPB - 1)], ids, mask=m)
        return cnt + jnp.sum(mi)

    cnt = lax.fori_loop(0, BPT // L, scan_body, jnp.int32(0))
    cnt = jnp.minimum(cnt, jnp.int32(CAPB))

    lcnt_smem[0] = 0

    def chunk_body(c, _):
        @pl.when(c * L < cnt)
        def _():
            bidv = idbuf[pl.ds(pl.multiple_of(c * L, 8), L)]
            for r in range(L):
                @pl.when(c * L + r < cnt)
                def _():
                    bid = jnp.sum(jnp.where(lanes == r, bidv, 0))
                    grow = bid >> 2
                    gcol = (bid & (BPR - 1)) * BLK
                    pltpu.async_copy(
                        grad_ref.at[grow, pl.ds(gcol, BLK)],
                        blk.at[r], sem_blk).wait()

                    def grp_body(g, lc):
                        raw = blk[r, pl.ds(pl.multiple_of(g * L, 8), L)]
                        bits = lax.bitcast_convert_type(raw, jnp.int32)
                        key = bits & 0x7FFFFFFF
                        pos = bid * BLK + g * L + lanes
                        row = pos >> 12
                        col = pos & (N - 1)
                        sel = (col > row) & (key >= t_key)
                        si = sel.astype(jnp.int32)
                        wpos = jnp.minimum(lc + plsc.cumsum(si) - si,
                                           CAPC - 1)
                        plsc.store_scatter(lraw, [wpos], bits, mask=sel)
                        plsc.store_scatter(lidx, [wpos], pos, mask=sel)
                        return lc + jnp.sum(si)

                    lc = lax.fori_loop(0, BLK // L, grp_body, lcnt_smem[0])
                    lcnt_smem[0] = jnp.minimum(lc, jnp.int32(CAPC))
        return 0

    lax.fori_loop(0, CAPB // L, chunk_body, 0)
    lcnt = lcnt_smem[0]

    lpad = (lcnt + (L - 1)) & ~(L - 1)

    @pl.when(lpad > lcnt)
    def _():
        toff = pl.multiple_of(lpad - L, 8)
        tail = lraw[pl.ds(toff, L)]
        m = (lpad - L + lanes) < lcnt
        lraw[pl.ds(toff, L)] = jnp.where(m, tail, 0)
        tidx = lidx[pl.ds(toff, L)]
        lidx[pl.ds(toff, L)] = jnp.where(m, tidx, 0)

    off = plsc.fetch_and_add(cnt_smem.at[0], lpad, subcore_id=0)
    off = jnp.minimum(off, jnp.int32(SPC - CAPC))

    def pack_body(q, _):
        @pl.when(q * L < lpad)
        def _():
            poff = pl.multiple_of(off + q * L, 8)
            pltpu.sync_copy(lraw.at[pl.ds(q * L, L)],
                            sp_raw.at[pl.ds(poff, L)])
            pltpu.sync_copy(lidx.at[pl.ds(q * L, L)],
                            sp_idx.at[pl.ds(poff, L)])
        return 0

    lax.fori_loop(0, CAPC // L, pack_body, 0)
    plsc.subcore_barrier()

    pltpu.sync_copy(sp_raw, keys)

    def count_ge(x):
        def cbody(q, acc):
            kk = keys[pl.ds(pl.multiple_of(q * L, 8), L)] & 0x7FFFFFFF
            return acc + jnp.sum((kk >= x).astype(jnp.int32))
        return lax.fori_loop(0, SPC // L, cbody, jnp.int32(0))

    def bis_body(_, lohi):
        lo, hi = lohi
        mid = lo + (hi - lo) // 2
        ok = count_ge(mid) >= k_target
        return jnp.where(ok, mid, lo), jnp.where(ok, hi, mid)

    tfin, _ = lax.fori_loop(0, 31, bis_body, (t_key, IMAX))

    pltpu.sync_copy(sp_idx, kidx)
    base = sid * SLICE

    for gq in range(SLICE // L):
        goff = pl.multiple_of(base + gq * L, 8)
        raw = keys[pl.ds(goff, L)]
        idx = kidx[pl.ds(goff, L)]
        key = raw & 0x7FFFFFFF
        sel = (key >= tfin) & ((idx & 1) == cid)

        @pl.when(jnp.sum(sel.astype(jnp.int32)) > 0)
        def _(raw=raw, idx=idx, sel=sel):
            fl = plsc.all_reduce_ffs(sel)
            first_idx = jnp.sum(jnp.where(lanes == fl, idx, 0))
            raw_first = jnp.sum(jnp.where(lanes == fl, raw, 0))
            idx_s = jnp.where(sel, idx, first_idx)
            raw_s = jnp.where(sel, raw, raw_first)
            u = idx_s >> 12
            v = idx_s & (N - 1)
            idx_t = (v << 12) | u
            pltpu.async_copy(anew_ref.at[idx_s], avbuf, sem_edge).wait()
            av = avbuf[...]
            gv = lax.bitcast_convert_type(raw_s, jnp.float32)
            exist = av > 0.0
            dec = exist & (gv <= 0.0)
            inc = (~exist) & (gv >= 0.0)
            d = jnp.where(dec, jnp.float32(-STEP),
                          jnp.where(inc, jnp.float32(STEP), jnp.float32(0.0)))
            valbuf[...] = av + d
            pltpu.async_copy(valbuf, anew_ref.at[idx_t], sem_edge).wait()
            pltpu.async_copy(valbuf, anew_ref.at[idx_s], sem_edge).wait()

    @pl.when((sid == 0) & (cid == 0))
    def _():
        stat[pl.ds(0, L)] = jnp.full((L,), tfin, jnp.int32)
        pltpu.sync_copy(stat.at[pl.ds(0, L)], out_ref)


@jax.jit
def _impl(gradA, A_logits, topk):
    maxes, a_new = pl.pallas_call(
        _maxes_copy_kernel,
        grid=(NBANDS,),
        in_specs=[
            pl.BlockSpec((BAND, N), lambda b: (b, 0)),
            pl.BlockSpec((BAND, N), lambda b: (b, 0)),
        ],
        out_specs=[
            pl.BlockSpec((BAND * BPR // 128, 128), lambda b: (b, 0)),
            pl.BlockSpec((BAND * N // 128, 128), lambda b: (b, 0)),
        ],
        out_shape=[
            jax.ShapeDtypeStruct((NBLK // 128, 128), jnp.float32),
            jax.ShapeDtypeStruct((NSQ // 128, 128), jnp.float32),
        ],
    )(gradA, A_logits)

    topk_arr = jnp.asarray(topk, jnp.int32).reshape((1,))
    tk = pl.pallas_call(
        _thresh_kernel,
        in_specs=[
            pl.BlockSpec(memory_space=pltpu.SMEM),
            pl.BlockSpec((NBLK // 128, 128), lambda: (0, 0)),
        ],
        out_specs=pl.BlockSpec(memory_space=pltpu.SMEM),
        out_shape=jax.ShapeDtypeStruct((L,), jnp.int32),
    )(topk_arr, maxes)

    sc = pl.kernel(
        _sc_body,
        out_type=jax.ShapeDtypeStruct((L,), jnp.int32),
        mesh=plsc.VectorSubcoreMesh(
            core_axis_name="c", subcore_axis_name="s",
            num_cores=NCORE, num_subcores=NSUB),
        compiler_params=pltpu.CompilerParams(needs_layout_passes=False),
        scratch_types=[
            pltpu.VMEM((BPT,), jnp.float32),
            pltpu.VMEM((CAPB,), jnp.int32),
            pltpu.VMEM((L, BLK), jnp.float32),
            pltpu.VMEM((CAPC,), jnp.int32),
            pltpu.VMEM((CAPC,), jnp.int32),
            pltpu.VMEM((SPC,), jnp.int32),
            pltpu.VMEM((SPC,), jnp.int32),
            pltpu.VMEM((L,), jnp.int32),
            pltpu.VMEM((L,), jnp.int32),
            pltpu.VMEM((L,), jnp.float32),
            pltpu.VMEM((L,), jnp.float32),
            pltpu.VMEM_SHARED((SPC,), jnp.int32),
            pltpu.VMEM_SHARED((SPC,), jnp.int32),
            pltpu.SMEM((1,), jnp.int32),
            pltpu.SMEM((1,), jnp.int32),
            pltpu.SemaphoreType.DMA,
            pltpu.SemaphoreType.DMA,
        ],
    )
    mx_flat = maxes.reshape(NBLK)
    anew_ref = jax.new_ref(a_new.reshape(NSQ))
    _ = sc(gradA, mx_flat, tk, anew_ref)
    return anew_ref[...].reshape(N, N)


def kernel(gradA, A_logits, topk):
    return _impl(gradA, A_logits, topk)


# speedup vs baseline: 1.5967x; 1.3907x over previous
import functools

import jax
import jax.numpy as jnp
import numpy as np
from jax import lax
from jax.experimental import pallas as pl
from jax.experimental.pallas import tpu as pltpu
from jax.experimental.pallas import tpu_sc as plsc

N = 4096
NSQ = N * N
BAND = 256
NBANDS = N // BAND
BLK = 128
NBLK = NSQ // BLK
BPR = N // BLK
STEP = 2.5
MAXK = 1024

NCORE = 2
NSUB = 16
L = 16
BPT = NBLK // NSUB
CAPB = 256
CAPC = 256
SPC = 2048
SLICE = SPC // NSUB
IMIN = np.int32(-2147483648)
IMAX = np.int32(2147483647)


def _maxes_copy_kernel(g_ref, a_ref, m_ref, out_ref):
    b = pl.program_id(0)
    g = g_ref[...]
    rows = lax.broadcasted_iota(jnp.int32, (BAND, N), 0) + b * BAND
    cols = lax.broadcasted_iota(jnp.int32, (BAND, N), 1)
    s = jnp.where(cols > rows, jnp.abs(g), 0.0)
    m = jnp.max(s.reshape(BAND, BPR, BLK), axis=-1)
    m_ref[...] = m.reshape(BAND * BPR // 128, 128)
    a = jnp.where(cols == rows, jnp.float32(-10.0), a_ref[...])
    out_ref[...] = a.reshape(BAND * N // 128, 128)


def _thresh_kernel(topk_ref, m_ref, t_ref):
    keys = lax.bitcast_convert_type(m_ref[...], jnp.int32)
    target = jnp.minimum(topk_ref[0], jnp.int32(MAXK))

    def body(_, lohi):
        lo, hi = lohi
        mid = lo + (hi - lo) // 2
        c = jnp.sum((keys >= mid).astype(jnp.int32))
        ok = c >= target
        return jnp.where(ok, mid, lo), jnp.where(ok, hi, mid)

    lo, _ = lax.fori_loop(0, 31, body, (jnp.int32(0), IMAX))
    mmax = jnp.max(keys) + 1
    for i in range(L):
        t_ref[i] = jnp.where(i == 1, target, jnp.where(i == 2, mmax, lo))


def _lane_iota():
    return lax.iota(jnp.int32, L)


def _sc_body(grad_ref, mx_ref, tk_ref, anew_ref, out_ref,
             mxv, idbuf, blk, lraw, lidx, keys, kidx, tkv, stat,
             avbuf, valbuf,
             sp_raw, sp_idx, cnt_smem, lcnt_smem, sem_blk, sem_edge):
    sid = lax.axis_index("s")
    cid = lax.axis_index("c")
    lanes = _lane_iota()
    zi = jnp.zeros((L,), jnp.int32)

    @pl.when(sid == 0)
    def _():
        cnt_smem[0] = 0
    stat[pl.ds(0, L)] = zi
    for q in range(SLICE // L):
        zoff = pl.multiple_of(sid * SLICE + q * L, 8)
        pltpu.sync_copy(stat.at[pl.ds(0, L)], sp_raw.at[pl.ds(zoff, L)])
        pltpu.sync_copy(stat.at[pl.ds(0, L)], sp_idx.at[pl.ds(zoff, L)])
    for q in range(CAPB // L):
        idbuf[pl.ds(q * L, L)] = zi
    plsc.subcore_barrier()

    pltpu.sync_copy(mx_ref.at[pl.ds(pl.multiple_of(sid * BPT, 8), BPT)], mxv)
    pltpu.sync_copy(tk_ref, tkv)
    tk16 = tkv[...]
    t_key = jnp.sum(jnp.where(lanes == 0, tk16, 0))
    k_target = jnp.sum(jnp.where(lanes == 1, tk16, 0))
    hi_key = jnp.sum(jnp.where(lanes == 2, tk16, 0))

    def scan_body(g, cnt):
        m16 = mxv[pl.ds(pl.multiple_of(g * L, 8), L)]
        kk = lax.bitcast_convert_type(m16, jnp.int32)
        m = kk >= t_key
        ids = sid * BPT + g * L + lanes
        mi = m.astype(jnp.int32)
        wpos = cnt + plsc.cumsum(mi) - mi
        plsc.store_scatter(idbuf, [jnp.minimum(wpos, CAPB - 1)], ids, mask=m)
        return cnt + jnp.sum(mi)

    cnt = lax.fori_loop(0, BPT // L, scan_body, jnp.int32(0))
    cnt = jnp.minimum(cnt, jnp.int32(CAPB))

    lcnt_smem[0] = 0

    def chunk_body(c, _):
        @pl.when(c * L < cnt)
        def _():
            bidv = idbuf[pl.ds(pl.multiple_of(c * L, 8), L)]
            for r in range(L):
                @pl.when(c * L + r < cnt)
                def _():
                    bid = jnp.sum(jnp.where(lanes == r, bidv, 0))
                    grow = bid >> 5
                    gcol = (bid & (BPR - 1)) * BLK
                    pltpu.async_copy(
                        grad_ref.at[grow, pl.ds(gcol, BLK)],
                        blk.at[r], sem_blk).wait()

                    def grp_body(g, lc):
                        raw = blk[r, pl.ds(pl.multiple_of(g * L, 8), L)]
                        bits = lax.bitcast_convert_type(raw, jnp.int32)
                        key = bits & 0x7FFFFFFF
                        pos = bid * BLK + g * L + lanes
                        row = pos >> 12
                        col = pos & (N - 1)
                        sel = (col > row) & (key >= t_key)
                        si = sel.astype(jnp.int32)
                        wpos = jnp.minimum(lc + plsc.cumsum(si) - si,
                                           CAPC - 1)
                        plsc.store_scatter(lraw, [wpos], bits, mask=sel)
                        plsc.store_scatter(lidx, [wpos], pos, mask=sel)
                        return lc + jnp.sum(si)

                    lc = lax.fori_loop(0, BLK // L, grp_body, lcnt_smem[0])
                    lcnt_smem[0] = jnp.minimum(lc, jnp.int32(CAPC))
        return 0

    lax.fori_loop(0, CAPB // L, chunk_body, 0)
    lcnt = lcnt_smem[0]

    lpad = (lcnt + (L - 1)) & ~(L - 1)

    @pl.when(lpad > lcnt)
    def _():
        toff = pl.multiple_of(lpad - L, 8)
        tail = lraw[pl.ds(toff, L)]
        m = (lpad - L + lanes) < lcnt
        lraw[pl.ds(toff, L)] = jnp.where(m, tail, 0)
        tidx = lidx[pl.ds(toff, L)]
        lidx[pl.ds(toff, L)] = jnp.where(m, tidx, 0)

    off = plsc.fetch_and_add(cnt_smem.at[0], lpad, subcore_id=0)
    off = jnp.minimum(off, jnp.int32(SPC - CAPC))

    def pack_body(q, _):
        @pl.when(q * L < lpad)
        def _():
            poff = pl.multiple_of(off + q * L, 8)
            pltpu.sync_copy(lraw.at[pl.ds(q * L, L)],
                            sp_raw.at[pl.ds(poff, L)])
            pltpu.sync_copy(lidx.at[pl.ds(q * L, L)],
                            sp_idx.at[pl.ds(poff, L)])
        return 0

    lax.fori_loop(0, CAPC // L, pack_body, 0)
    plsc.subcore_barrier()

    pltpu.sync_copy(sp_raw, keys)
    total = plsc.fetch_and_add(cnt_smem.at[0], 0, subcore_id=0)
    ngrp = jnp.minimum((total + (L - 1)) >> 4, jnp.int32(SPC // L))

    def count_ge(x):
        def cbody(q, acc):
            kk = keys[pl.ds(pl.multiple_of(q * L, 8), L)] & 0x7FFFFFFF
            return acc + jnp.sum((kk >= x).astype(jnp.int32))
        return lax.fori_loop(0, ngrp, cbody, jnp.int32(0))

    def bis_body(_, lohi):
        lo, hi = lohi
        mid = lo + (hi - lo) // 2
        ok = count_ge(mid) >= k_target
        return jnp.where(ok, mid, lo), jnp.where(ok, hi, mid)

    tfin, _ = lax.fori_loop(0, 31, bis_body, (t_key, hi_key))

    pltpu.sync_copy(sp_idx, kidx)
    base = sid * SLICE

    for gq in range(SLICE // L):
        goff = pl.multiple_of(base + gq * L, 8)
        raw = keys[pl.ds(goff, L)]
        idx = kidx[pl.ds(goff, L)]
        key = raw & 0x7FFFFFFF
        sel = (key >= tfin) & ((idx & 1) == cid)

        @pl.when(jnp.sum(sel.astype(jnp.int32)) > 0)
        def _(raw=raw, idx=idx, sel=sel):
            fl = plsc.all_reduce_ffs(sel)
            first_idx = jnp.sum(jnp.where(lanes == fl, idx, 0))
            raw_first = jnp.sum(jnp.where(lanes == fl, raw, 0))
            idx_s = jnp.where(sel, idx, first_idx)
            raw_s = jnp.where(sel, raw, raw_first)
            u = idx_s >> 12
            v = idx_s & (N - 1)
            idx_t = (v << 12) | u
            pltpu.async_copy(anew_ref.at[idx_s], avbuf, sem_edge).wait()
            av = avbuf[...]
            gv = lax.bitcast_convert_type(raw_s, jnp.float32)
            exist = av > 0.0
            dec = exist & (gv <= 0.0)
            inc = (~exist) & (gv >= 0.0)
            d = jnp.where(dec, jnp.float32(-STEP),
                          jnp.where(inc, jnp.float32(STEP), jnp.float32(0.0)))
            valbuf[...] = av + d
            pltpu.async_copy(valbuf, anew_ref.at[idx_t], sem_edge).wait()
            pltpu.async_copy(valbuf, anew_ref.at[idx_s], sem_edge).wait()

    @pl.when((sid == 0) & (cid == 0))
    def _():
        stat[pl.ds(0, L)] = jnp.full((L,), tfin, jnp.int32)
        pltpu.sync_copy(stat.at[pl.ds(0, L)], out_ref)


@jax.jit
def _impl(gradA, A_logits, topk):
    maxes, a_new = pl.pallas_call(
        _maxes_copy_kernel,
        grid=(NBANDS,),
        in_specs=[
            pl.BlockSpec((BAND, N), lambda b: (b, 0)),
            pl.BlockSpec((BAND, N), lambda b: (b, 0)),
        ],
        out_specs=[
            pl.BlockSpec((BAND * BPR // 128, 128), lambda b: (b, 0)),
            pl.BlockSpec((BAND * N // 128, 128), lambda b: (b, 0)),
        ],
        out_shape=[
            jax.ShapeDtypeStruct((NBLK // 128, 128), jnp.float32),
            jax.ShapeDtypeStruct((NSQ // 128, 128), jnp.float32),
        ],
    )(gradA, A_logits)

    topk_arr = jnp.asarray(topk, jnp.int32).reshape((1,))
    tk = pl.pallas_call(
        _thresh_kernel,
        in_specs=[
            pl.BlockSpec(memory_space=pltpu.SMEM),
            pl.BlockSpec((NBLK // 128, 128), lambda: (0, 0)),
        ],
        out_specs=pl.BlockSpec(memory_space=pltpu.SMEM),
        out_shape=jax.ShapeDtypeStruct((L,), jnp.int32),
    )(topk_arr, maxes)

    sc = pl.kernel(
        _sc_body,
        out_type=jax.ShapeDtypeStruct((L,), jnp.int32),
        mesh=plsc.VectorSubcoreMesh(
            core_axis_name="c", subcore_axis_name="s",
            num_cores=NCORE, num_subcores=NSUB),
        compiler_params=pltpu.CompilerParams(needs_layout_passes=False),
        scratch_types=[
            pltpu.VMEM((BPT,), jnp.float32),
            pltpu.VMEM((CAPB,), jnp.int32),
            pltpu.VMEM((L, BLK), jnp.float32),
            pltpu.VMEM((CAPC,), jnp.int32),
            pltpu.VMEM((CAPC,), jnp.int32),
            pltpu.VMEM((SPC,), jnp.int32),
            pltpu.VMEM((SPC,), jnp.int32),
            pltpu.VMEM((L,), jnp.int32),
            pltpu.VMEM((L,), jnp.int32),
            pltpu.VMEM((L,), jnp.float32),
            pltpu.VMEM((L,), jnp.float32),
            pltpu.VMEM_SHARED((SPC,), jnp.int32),
            pltpu.VMEM_SHARED((SPC,), jnp.int32),
            pltpu.SMEM((1,), jnp.int32),
            pltpu.SMEM((1,), jnp.int32),
            pltpu.SemaphoreType.DMA,
            pltpu.SemaphoreType.DMA,
        ],
    )
    mx_flat = maxes.reshape(NBLK)
    anew_ref = jax.new_ref(a_new.reshape(NSQ))
    _ = sc(gradA, mx_flat, tk, anew_ref)
    return anew_ref[...].reshape(N, N)


def kernel(gradA, A_logits, topk):
    return _impl(gradA, A_logits, topk)


# speedup vs baseline: 1.9614x; 1.2284x over previous
import functools

import jax
import jax.numpy as jnp
import numpy as np
from jax import lax
from jax.experimental import pallas as pl
from jax.experimental.pallas import tpu as pltpu
from jax.experimental.pallas import tpu_sc as plsc

N = 4096
NSQ = N * N
BAND = 256
NBANDS = N // BAND
BLK = 128
NBLK = NSQ // BLK
BPR = N // BLK
STEP = 2.5
MAXK = 1024

NCORE = 2
NSUB = 16
L = 16
BPT = NBLK // NSUB
CAPB = 256
CAPC = 256
SPC = 2048
SLICE = SPC // NSUB
IMIN = np.int32(-2147483648)
IMAX = np.int32(2147483647)


def _maxes_copy_kernel(g_ref, a_ref, m_ref, out_ref):
    b = pl.program_id(0)
    g = g_ref[...]
    rows = lax.broadcasted_iota(jnp.int32, (BAND, N), 0) + b * BAND
    cols = lax.broadcasted_iota(jnp.int32, (BAND, N), 1)
    s = jnp.where(cols > rows, jnp.abs(g), 0.0)
    m = jnp.max(s.reshape(BAND, BPR, BLK), axis=-1)
    m_ref[...] = m.reshape(BAND * BPR // 128, 128)
    a = jnp.where(cols == rows, jnp.float32(-10.0), a_ref[...])
    out_ref[...] = a.reshape(BAND * N // 128, 128)


def _thresh_kernel(topk_ref, m_ref, t_ref):
    keys = lax.bitcast_convert_type(m_ref[...], jnp.int32)
    target = jnp.minimum(topk_ref[0], jnp.int32(MAXK))

    def body(_, lohi):
        lo, hi = lohi
        mid = lo + (hi - lo) // 2
        c = jnp.sum((keys >= mid).astype(jnp.int32))
        ok = c >= target
        return jnp.where(ok, mid, lo), jnp.where(ok, hi, mid)

    lo, _ = lax.fori_loop(0, 31, body, (jnp.int32(0), IMAX))
    mmax = jnp.max(keys) + 1
    for i in range(L):
        t_ref[i] = jnp.where(i == 1, target, jnp.where(i == 2, mmax, lo))


def _lane_iota():
    return lax.iota(jnp.int32, L)


def _sc_body(grad_ref, mx_ref, tk_ref, anew_ref, out_ref,
             mxv, idbuf, blk, lraw, lidx, keys, kidx, tkv, stat,
             avbuf, valbuf,
             sp_raw, sp_idx, cnt_smem, lcnt_smem, sem_blk, sem_edge):
    sid = lax.axis_index("s")
    cid = lax.axis_index("c")
    lanes = _lane_iota()
    zi = jnp.zeros((L,), jnp.int32)

    @pl.when(sid == 0)
    def _():
        cnt_smem[0] = 0
    for q in range(SLICE // L):
        keys[pl.ds(q * L, L)] = zi
    zoff = pl.multiple_of(sid * SLICE, 8)
    pltpu.sync_copy(keys.at[pl.ds(0, SLICE)], sp_raw.at[pl.ds(zoff, SLICE)])
    pltpu.sync_copy(keys.at[pl.ds(0, SLICE)], sp_idx.at[pl.ds(zoff, SLICE)])
    for q in range(CAPB // L):
        idbuf[pl.ds(q * L, L)] = zi
    plsc.subcore_barrier()

    pltpu.sync_copy(mx_ref.at[pl.ds(pl.multiple_of(sid * BPT, 8), BPT)], mxv)
    pltpu.sync_copy(tk_ref, tkv)
    tk16 = tkv[...]
    t_key = jnp.sum(jnp.where(lanes == 0, tk16, 0))
    k_target = jnp.sum(jnp.where(lanes == 1, tk16, 0))
    hi_key = jnp.sum(jnp.where(lanes == 2, tk16, 0))

    def scan_body(g, cnt):
        m16 = mxv[pl.ds(pl.multiple_of(g * L, 8), L)]
        kk = lax.bitcast_convert_type(m16, jnp.int32)
        m = kk >= t_key
        ids = sid * BPT + g * L + lanes
        mi = m.astype(jnp.int32)
        wpos = cnt + plsc.cumsum(mi) - mi
        plsc.store_scatter(idbuf, [jnp.minimum(wpos, CAPB - 1)], ids, mask=m)
        return cnt + jnp.sum(mi)

    cnt = lax.fori_loop(0, BPT // L, scan_body, jnp.int32(0))
    cnt = jnp.minimum(cnt, jnp.int32(CAPB))

    lcnt_smem[0] = 0

    def chunk_body(c, _):
        @pl.when(c * L < cnt)
        def _():
            bidv = idbuf[pl.ds(pl.multiple_of(c * L, 8), L)]
            for r in range(L):
                @pl.when(c * L + r < cnt)
                def _(r=r):
                    bid = jnp.sum(jnp.where(lanes == r, bidv, 0))
                    grow = bid >> 5
                    gcol = (bid & (BPR - 1)) * BLK
                    pltpu.async_copy(grad_ref.at[grow, pl.ds(gcol, BLK)],
                                     blk.at[r], sem_blk)
            for r in range(L):
                @pl.when(c * L + r < cnt)
                def _(r=r):
                    bid = jnp.sum(jnp.where(lanes == r, bidv, 0))
                    grow = bid >> 5
                    gcol = (bid & (BPR - 1)) * BLK
                    pltpu.make_async_copy(grad_ref.at[grow, pl.ds(gcol, BLK)],
                                          blk.at[r], sem_blk).wait()
            for r in range(L):
                @pl.when(c * L + r < cnt)
                def _(r=r):
                    bid = jnp.sum(jnp.where(lanes == r, bidv, 0))

                    def grp_body(g, lc):
                        raw = blk[r, pl.ds(pl.multiple_of(g * L, 8), L)]
                        bits = lax.bitcast_convert_type(raw, jnp.int32)
                        key = bits & 0x7FFFFFFF
                        pos = bid * BLK + g * L + lanes
                        row = pos >> 12
                        col = pos & (N - 1)
                        sel = (col > row) & (key >= t_key)
                        si = sel.astype(jnp.int32)
                        wpos = jnp.minimum(lc + plsc.cumsum(si) - si,
                                           CAPC - 1)
                        plsc.store_scatter(lraw, [wpos], bits, mask=sel)
                        plsc.store_scatter(lidx, [wpos], pos, mask=sel)
                        return lc + jnp.sum(si)

                    lc = lax.fori_loop(0, BLK // L, grp_body, lcnt_smem[0])
                    lcnt_smem[0] = jnp.minimum(lc, jnp.int32(CAPC))
        return 0

    lax.fori_loop(0, CAPB // L, chunk_body, 0)
    lcnt = lcnt_smem[0]

    lpad = (lcnt + (L - 1)) & ~(L - 1)

    @pl.when(lpad > lcnt)
    def _():
        toff = pl.multiple_of(lpad - L, 8)
        tail = lraw[pl.ds(toff, L)]
        m = (lpad - L + lanes) < lcnt
        lraw[pl.ds(toff, L)] = jnp.where(m, tail, 0)
        tidx = lidx[pl.ds(toff, L)]
        lidx[pl.ds(toff, L)] = jnp.where(m, tidx, 0)

    off = plsc.fetch_and_add(cnt_smem.at[0], lpad, subcore_id=0)
    off = jnp.minimum(off, jnp.int32(SPC - CAPC))

    def pack_body(q, _):
        @pl.when(q * L < lpad)
        def _():
            poff = pl.multiple_of(off + q * L, 8)
            pltpu.sync_copy(lraw.at[pl.ds(q * L, L)],
                            sp_raw.at[pl.ds(poff, L)])
            pltpu.sync_copy(lidx.at[pl.ds(q * L, L)],
                            sp_idx.at[pl.ds(poff, L)])
        return 0

    lax.fori_loop(0, CAPC // L, pack_body, 0)
    plsc.subcore_barrier()

    pltpu.sync_copy(sp_raw, keys)
    total = plsc.fetch_and_add(cnt_smem.at[0], 0, subcore_id=0)
    ngrp = jnp.minimum((total + (L - 1)) >> 4, jnp.int32(SPC // L))

    def count_ge(x):
        def cbody(q, acc):
            kk = keys[pl.ds(pl.multiple_of(q * L, 8), L)] & 0x7FFFFFFF
            return acc + jnp.sum((kk >= x).astype(jnp.int32))
        return lax.fori_loop(0, ngrp, cbody, jnp.int32(0))

    def bis_body(_, lohi):
        lo, hi = lohi
        mid = lo + (hi - lo) // 2
        ok = count_ge(mid) >= k_target
        return jnp.where(ok, mid, lo), jnp.where(ok, hi, mid)

    tfin, _ = lax.fori_loop(0, 31, bis_body, (t_key, hi_key))

    pltpu.sync_copy(sp_idx, kidx)
    base = sid * SLICE

    for gq in range(SLICE // L):
        goff = pl.multiple_of(base + gq * L, 8)
        raw = keys[pl.ds(goff, L)]
        idx = kidx[pl.ds(goff, L)]
        key = raw & 0x7FFFFFFF
        sel = (key >= tfin) & ((idx & 1) == cid)

        @pl.when(jnp.sum(sel.astype(jnp.int32)) > 0)
        def _(raw=raw, idx=idx, sel=sel):
            fl = plsc.all_reduce_ffs(sel)
            first_idx = jnp.sum(jnp.where(lanes == fl, idx, 0))
            raw_first = jnp.sum(jnp.where(lanes == fl, raw, 0))
            idx_s = jnp.where(sel, idx, first_idx)
            raw_s = jnp.where(sel, raw, raw_first)
            u = idx_s >> 12
            v = idx_s & (N - 1)
            idx_t = (v << 12) | u
            pltpu.async_copy(anew_ref.at[idx_s], avbuf, sem_edge).wait()
            av = avbuf[...]
            gv = lax.bitcast_convert_type(raw_s, jnp.float32)
            exist = av > 0.0
            dec = exist & (gv <= 0.0)
            inc = (~exist) & (gv >= 0.0)
            d = jnp.where(dec, jnp.float32(-STEP),
                          jnp.where(inc, jnp.float32(STEP), jnp.float32(0.0)))
            valbuf[...] = av + d
            pltpu.async_copy(valbuf, anew_ref.at[idx_t], sem_edge).wait()
            pltpu.async_copy(valbuf, anew_ref.at[idx_s], sem_edge).wait()

    @pl.when((sid == 0) & (cid == 0))
    def _():
        stat[pl.ds(0, L)] = jnp.full((L,), tfin, jnp.int32)
        pltpu.sync_copy(stat.at[pl.ds(0, L)], out_ref)


@jax.jit
def _impl(gradA, A_logits, topk):
    maxes, a_new = pl.pallas_call(
        _maxes_copy_kernel,
        grid=(NBANDS,),
        in_specs=[
            pl.BlockSpec((BAND, N), lambda b: (b, 0)),
            pl.BlockSpec((BAND, N), lambda b: (b, 0)),
        ],
        out_specs=[
            pl.BlockSpec((BAND * BPR // 128, 128), lambda b: (b, 0)),
            pl.BlockSpec((BAND * N // 128, 128), lambda b: (b, 0)),
        ],
        out_shape=[
            jax.ShapeDtypeStruct((NBLK // 128, 128), jnp.float32),
            jax.ShapeDtypeStruct((NSQ // 128, 128), jnp.float32),
        ],
    )(gradA, A_logits)

    topk_arr = jnp.asarray(topk, jnp.int32).reshape((1,))
    tk = pl.pallas_call(
        _thresh_kernel,
        in_specs=[
            pl.BlockSpec(memory_space=pltpu.SMEM),
            pl.BlockSpec((NBLK // 128, 128), lambda: (0, 0)),
        ],
        out_specs=pl.BlockSpec(memory_space=pltpu.SMEM),
        out_shape=jax.ShapeDtypeStruct((L,), jnp.int32),
    )(topk_arr, maxes)

    sc = pl.kernel(
        _sc_body,
        out_type=jax.ShapeDtypeStruct((L,), jnp.int32),
        mesh=plsc.VectorSubcoreMesh(
            core_axis_name="c", subcore_axis_name="s",
            num_cores=NCORE, num_subcores=NSUB),
        compiler_params=pltpu.CompilerParams(needs_layout_passes=False),
        scratch_types=[
            pltpu.VMEM((BPT,), jnp.float32),
            pltpu.VMEM((CAPB,), jnp.int32),
            pltpu.VMEM((L, BLK), jnp.float32),
            pltpu.VMEM((CAPC,), jnp.int32),
            pltpu.VMEM((CAPC,), jnp.int32),
            pltpu.VMEM((SPC,), jnp.int32),
            pltpu.VMEM((SPC,), jnp.int32),
            pltpu.VMEM((L,), jnp.int32),
            pltpu.VMEM((L,), jnp.int32),
            pltpu.VMEM((L,), jnp.float32),
            pltpu.VMEM((L,), jnp.float32),
            pltpu.VMEM_SHARED((SPC,), jnp.int32),
            pltpu.VMEM_SHARED((SPC,), jnp.int32),
            pltpu.SMEM((1,), jnp.int32),
            pltpu.SMEM((1,), jnp.int32),
            pltpu.SemaphoreType.DMA,
            pltpu.SemaphoreType.DMA,
        ],
    )
    mx_flat = maxes.reshape(NBLK)
    anew_ref = jax.new_ref(a_new.reshape(NSQ))
    _ = sc(gradA, mx_flat, tk, anew_ref)
    return anew_ref[...].reshape(N, N)


def kernel(gradA, A_logits, topk):
    return _impl(gradA, A_logits, topk)


# speedup vs baseline: 1.9655x; 1.0021x over previous
import functools

import jax
import jax.numpy as jnp
import numpy as np
from jax import lax
from jax.experimental import pallas as pl
from jax.experimental.pallas import tpu as pltpu
from jax.experimental.pallas import tpu_sc as plsc

N = 4096
NSQ = N * N
BAND = 256
NBANDS = N // BAND
BLK = 128
NBLK = NSQ // BLK
BPR = N // BLK
STEP = 2.5
MAXK = 1024

NCORE = 2
NSUB = 16
L = 16
BPT = NBLK // NSUB
CAPB = 256
CAPC = 256
SPC = 2048
SLICE = SPC // NSUB
IMIN = np.int32(-2147483648)
IMAX = np.int32(2147483647)


def _maxes_copy_kernel(g_ref, a_ref, m_ref, out_ref):
    b = pl.program_id(0)
    g = g_ref[...]
    rows = lax.broadcasted_iota(jnp.int32, (BAND, N), 0) + b * BAND
    cols = lax.broadcasted_iota(jnp.int32, (BAND, N), 1)
    s = jnp.where(cols > rows, jnp.abs(g), 0.0)
    m = jnp.max(s.reshape(BAND, BPR, BLK), axis=-1)
    m_ref[...] = m.reshape(BAND * BPR // 128, 128)
    a = jnp.where(cols == rows, jnp.float32(-10.0), a_ref[...])
    out_ref[...] = a.reshape(BAND * N // 128, 128)


def _thresh_kernel(topk_ref, m_ref, t_ref):
    keys = lax.bitcast_convert_type(m_ref[...], jnp.int32)
    target = jnp.minimum(topk_ref[0], jnp.int32(MAXK))

    def body(_, lohi):
        lo, hi = lohi
        mid = lo + (hi - lo) // 2
        c = jnp.sum((keys >= mid).astype(jnp.int32))
        ok = c >= target
        return jnp.where(ok, mid, lo), jnp.where(ok, hi, mid)

    lo, _ = lax.fori_loop(0, 31, body, (jnp.int32(0), IMAX))
    mmax = jnp.max(keys) + 1
    for i in range(L):
        t_ref[i] = jnp.where(i == 1, target, jnp.where(i == 2, mmax, lo))


def _lane_iota():
    return lax.iota(jnp.int32, L)


def _sc_body(grad_ref, mx_ref, tk_ref, anew_ref, out_ref,
             mxv, idbuf, blk, lraw, lidx, keys, kidx, tkv, stat,
             sidx, sidxt, sraw, av128, sval,
             sp_raw, sp_idx, cnt_smem, lcnt_smem, aux_smem,
             sem_blk, sem_edge):
    sid = lax.axis_index("s")
    cid = lax.axis_index("c")
    lanes = _lane_iota()
    zi = jnp.zeros((L,), jnp.int32)

    @pl.when(sid == 0)
    def _():
        cnt_smem[0] = 0
    for q in range(SLICE // L):
        keys[pl.ds(q * L, L)] = zi
    zoff = pl.multiple_of(sid * SLICE, 8)
    pltpu.sync_copy(keys.at[pl.ds(0, SLICE)], sp_raw.at[pl.ds(zoff, SLICE)])
    pltpu.sync_copy(keys.at[pl.ds(0, SLICE)], sp_idx.at[pl.ds(zoff, SLICE)])
    for q in range(CAPB // L):
        idbuf[pl.ds(q * L, L)] = zi
    plsc.subcore_barrier()

    pltpu.sync_copy(mx_ref.at[pl.ds(pl.multiple_of(sid * BPT, 8), BPT)], mxv)
    pltpu.sync_copy(tk_ref, tkv)
    tk16 = tkv[...]
    t_key = jnp.sum(jnp.where(lanes == 0, tk16, 0))
    k_target = jnp.sum(jnp.where(lanes == 1, tk16, 0))
    hi_key = jnp.sum(jnp.where(lanes == 2, tk16, 0))

    def scan_body(g, cnt):
        m16 = mxv[pl.ds(pl.multiple_of(g * L, 8), L)]
        kk = lax.bitcast_convert_type(m16, jnp.int32)
        m = kk >= t_key
        ids = sid * BPT + g * L + lanes
        mi = m.astype(jnp.int32)
        wpos = cnt + plsc.cumsum(mi) - mi
        plsc.store_scatter(idbuf, [jnp.minimum(wpos, CAPB - 1)], ids, mask=m)
        return cnt + jnp.sum(mi)

    cnt = lax.fori_loop(0, BPT // L, scan_body, jnp.int32(0))
    cnt = jnp.minimum(cnt, jnp.int32(CAPB))

    lcnt_smem[0] = 0

    def chunk_body(c, _):
        @pl.when(c * L < cnt)
        def _():
            bidv = idbuf[pl.ds(pl.multiple_of(c * L, 8), L)]
            for r in range(L):
                @pl.when(c * L + r < cnt)
                def _(r=r):
                    bid = jnp.sum(jnp.where(lanes == r, bidv, 0))
                    grow = bid >> 5
                    gcol = (bid & (BPR - 1)) * BLK
                    pltpu.async_copy(grad_ref.at[grow, pl.ds(gcol, BLK)],
                                     blk.at[r], sem_blk)
            for r in range(L):
                @pl.when(c * L + r < cnt)
                def _(r=r):
                    bid = jnp.sum(jnp.where(lanes == r, bidv, 0))
                    grow = bid >> 5
                    gcol = (bid & (BPR - 1)) * BLK
                    pltpu.make_async_copy(grad_ref.at[grow, pl.ds(gcol, BLK)],
                                          blk.at[r], sem_blk).wait()
            for r in range(L):
                @pl.when(c * L + r < cnt)
                def _(r=r):
                    bid = jnp.sum(jnp.where(lanes == r, bidv, 0))

                    def grp_body(g, lc):
                        raw = blk[r, pl.ds(pl.multiple_of(g * L, 8), L)]
                        bits = lax.bitcast_convert_type(raw, jnp.int32)
                        key = bits & 0x7FFFFFFF
                        pos = bid * BLK + g * L + lanes
                        row = pos >> 12
                        col = pos & (N - 1)
                        sel = (col > row) & (key >= t_key)
                        si = sel.astype(jnp.int32)
                        wpos = jnp.minimum(lc + plsc.cumsum(si) - si,
                                           CAPC - 1)
                        plsc.store_scatter(lraw, [wpos], bits, mask=sel)
                        plsc.store_scatter(lidx, [wpos], pos, mask=sel)
                        return lc + jnp.sum(si)

                    lc = lax.fori_loop(0, BLK // L, grp_body, lcnt_smem[0])
                    lcnt_smem[0] = jnp.minimum(lc, jnp.int32(CAPC))
        return 0

    lax.fori_loop(0, CAPB // L, chunk_body, 0)
    lcnt = lcnt_smem[0]

    lpad = (lcnt + (L - 1)) & ~(L - 1)

    @pl.when(lpad > lcnt)
    def _():
        toff = pl.multiple_of(lpad - L, 8)
        tail = lraw[pl.ds(toff, L)]
        m = (lpad - L + lanes) < lcnt
        lraw[pl.ds(toff, L)] = jnp.where(m, tail, 0)
        tidx = lidx[pl.ds(toff, L)]
        lidx[pl.ds(toff, L)] = jnp.where(m, tidx, 0)

    off = plsc.fetch_and_add(cnt_smem.at[0], lpad, subcore_id=0)
    off = jnp.minimum(off, jnp.int32(SPC - CAPC))

    def pack_body(q, _):
        @pl.when(q * L < lpad)
        def _():
            poff = pl.multiple_of(off + q * L, 8)
            pltpu.sync_copy(lraw.at[pl.ds(q * L, L)],
                            sp_raw.at[pl.ds(poff, L)])
            pltpu.sync_copy(lidx.at[pl.ds(q * L, L)],
                            sp_idx.at[pl.ds(poff, L)])
        return 0

    lax.fori_loop(0, CAPC // L, pack_body, 0)
    plsc.subcore_barrier()

    pltpu.sync_copy(sp_raw, keys)
    total = plsc.fetch_and_add(cnt_smem.at[0], 0, subcore_id=0)
    ngrp = jnp.minimum((total + (L - 1)) >> 4, jnp.int32(SPC // L))

    def count_ge(x):
        def cbody(q, acc):
            kk = keys[pl.ds(pl.multiple_of(q * L, 8), L)] & 0x7FFFFFFF
            return acc + jnp.sum((kk >= x).astype(jnp.int32))
        return lax.fori_loop(0, ngrp, cbody, jnp.int32(0))

    def bis_body(_, lohi):
        lo, hi = lohi
        mid = lo + (hi - lo) // 2
        ok = count_ge(mid) >= k_target
        return jnp.where(ok, mid, lo), jnp.where(ok, hi, mid)

    tfin, _ = lax.fori_loop(0, 31, bis_body, (t_key, hi_key))

    pltpu.sync_copy(sp_idx, kidx)
    base = sid * SLICE

    aux_smem[1] = -1

    for gq in range(SLICE // L):
        goff = pl.multiple_of(base + gq * L, 8)
        raw = keys[pl.ds(goff, L)]
        idx = kidx[pl.ds(goff, L)]
        key = raw & 0x7FFFFFFF
        sel = (key >= tfin) & ((idx & 1) == cid)
        nsel = jnp.sum(sel.astype(jnp.int32))

        @pl.when(nsel > 0)
        def _(raw=raw, idx=idx, sel=sel, gq=gq):
            fl = plsc.all_reduce_ffs(sel)
            first_idx = jnp.sum(jnp.where(lanes == fl, idx, 0))
            raw_first = jnp.sum(jnp.where(lanes == fl, raw, 0))

            @pl.when(aux_smem[1] < 0)
            def _():
                aux_smem[1] = first_idx
                aux_smem[2] = raw_first
            sidx[pl.ds(gq * L, L)] = jnp.where(sel, idx, first_idx)
            sraw[pl.ds(gq * L, L)] = jnp.where(sel, raw, raw_first)

        @pl.when(nsel == 0)
        def _(gq=gq):
            sidx[pl.ds(gq * L, L)] = jnp.full((L,), -1, jnp.int32)

    @pl.when(aux_smem[1] >= 0)
    def _():
        ffi = aux_smem[1]
        ffr = aux_smem[2]
        for gq in range(SLICE // L):
            v = sidx[pl.ds(gq * L, L)]
            m = v < 0
            sidx[pl.ds(gq * L, L)] = jnp.where(m, ffi, v)
            w = sraw[pl.ds(gq * L, L)]
            sraw[pl.ds(gq * L, L)] = jnp.where(m, ffr, w)
        pltpu.async_copy(anew_ref.at[sidx], av128, sem_edge).wait()
        for gq in range(SLICE // L):
            av = av128[pl.ds(gq * L, L)]
            gv = lax.bitcast_convert_type(sraw[pl.ds(gq * L, L)], jnp.float32)
            exist = av > 0.0
            dec = exist & (gv <= 0.0)
            inc = (~exist) & (gv >= 0.0)
            d = jnp.where(dec, jnp.float32(-STEP),
                          jnp.where(inc, jnp.float32(STEP), jnp.float32(0.0)))
            sval[pl.ds(gq * L, L)] = av + d
            iv = sidx[pl.ds(gq * L, L)]
            u = iv >> 12
            v = iv & (N - 1)
            sidxt[pl.ds(gq * L, L)] = (v << 12) | u
        pltpu.async_copy(sval, anew_ref.at[sidx], sem_edge).wait()
        pltpu.async_copy(sval, anew_ref.at[sidxt], sem_edge).wait()

    @pl.when((sid == 0) & (cid == 0))
    def _():
        stat[pl.ds(0, L)] = jnp.full((L,), tfin, jnp.int32)
        pltpu.sync_copy(stat.at[pl.ds(0, L)], out_ref)


@jax.jit
def _impl(gradA, A_logits, topk):
    maxes, a_new = pl.pallas_call(
        _maxes_copy_kernel,
        grid=(NBANDS,),
        in_specs=[
            pl.BlockSpec((BAND, N), lambda b: (b, 0)),
            pl.BlockSpec((BAND, N), lambda b: (b, 0)),
        ],
        out_specs=[
            pl.BlockSpec((BAND * BPR // 128, 128), lambda b: (b, 0)),
            pl.BlockSpec((BAND * N // 128, 128), lambda b: (b, 0)),
        ],
        out_shape=[
            jax.ShapeDtypeStruct((NBLK // 128, 128), jnp.float32),
            jax.ShapeDtypeStruct((NSQ // 128, 128), jnp.float32),
        ],
    )(gradA, A_logits)

    topk_arr = jnp.asarray(topk, jnp.int32).reshape((1,))
    tk = pl.pallas_call(
        _thresh_kernel,
        in_specs=[
            pl.BlockSpec(memory_space=pltpu.SMEM),
            pl.BlockSpec((NBLK // 128, 128), lambda: (0, 0)),
        ],
        out_specs=pl.BlockSpec(memory_space=pltpu.SMEM),
        out_shape=jax.ShapeDtypeStruct((L,), jnp.int32),
    )(topk_arr, maxes)

    sc = pl.kernel(
        _sc_body,
        out_type=jax.ShapeDtypeStruct((L,), jnp.int32),
        mesh=plsc.VectorSubcoreMesh(
            core_axis_name="c", subcore_axis_name="s",
            num_cores=NCORE, num_subcores=NSUB),
        compiler_params=pltpu.CompilerParams(needs_layout_passes=False),
        scratch_types=[
            pltpu.VMEM((BPT,), jnp.float32),
            pltpu.VMEM((CAPB,), jnp.int32),
            pltpu.VMEM((L, BLK), jnp.float32),
            pltpu.VMEM((CAPC,), jnp.int32),
            pltpu.VMEM((CAPC,), jnp.int32),
            pltpu.VMEM((SPC,), jnp.int32),
            pltpu.VMEM((SPC,), jnp.int32),
            pltpu.VMEM((L,), jnp.int32),
            pltpu.VMEM((L,), jnp.int32),
            pltpu.VMEM((SLICE,), jnp.int32),
            pltpu.VMEM((SLICE,), jnp.int32),
            pltpu.VMEM((SLICE,), jnp.int32),
            pltpu.VMEM((SLICE,), jnp.float32),
            pltpu.VMEM((SLICE,), jnp.float32),
            pltpu.VMEM_SHARED((SPC,), jnp.int32),
            pltpu.VMEM_SHARED((SPC,), jnp.int32),
            pltpu.SMEM((1,), jnp.int32),
            pltpu.SMEM((1,), jnp.int32),
            pltpu.SMEM((4,), jnp.int32),
            pltpu.SemaphoreType.DMA,
            pltpu.SemaphoreType.DMA,
        ],
    )
    mx_flat = maxes.reshape(NBLK)
    anew_ref = jax.new_ref(a_new.reshape(NSQ))
    _ = sc(gradA, mx_flat, tk, anew_ref)
    return anew_ref[...].reshape(N, N)


def kernel(gradA, A_logits, topk):
    return _impl(gradA, A_logits, topk)


# speedup vs baseline: 1.9845x; 1.0097x over previous
import functools

import jax
import jax.numpy as jnp
import numpy as np
from jax import lax
from jax.experimental import pallas as pl
from jax.experimental.pallas import tpu as pltpu
from jax.experimental.pallas import tpu_sc as plsc

N = 4096
NSQ = N * N
BAND = 512
NBANDS = N // BAND
BLK = 128
NBLK = NSQ // BLK
BPR = N // BLK
STEP = 2.5
MAXK = 1024

NCORE = 2
NSUB = 16
L = 16
BPT = NBLK // NSUB
CAPB = 256
CAPC = 256
SPC = 2048
SLICE = SPC // NSUB
IMIN = np.int32(-2147483648)
IMAX = np.int32(2147483647)


def _maxes_copy_kernel(g_ref, a_ref, m_ref, out_ref):
    b = pl.program_id(0)
    g = g_ref[...]
    rows = lax.broadcasted_iota(jnp.int32, (BAND, N), 0) + b * BAND
    cols = lax.broadcasted_iota(jnp.int32, (BAND, N), 1)
    s = jnp.where(cols > rows, jnp.abs(g), 0.0)
    m = jnp.max(s.reshape(BAND, BPR, BLK), axis=-1)
    m_ref[...] = m.reshape(BAND * BPR // 128, 128)
    a = jnp.where(cols == rows, jnp.float32(-10.0), a_ref[...])
    out_ref[...] = a.reshape(BAND * N // 128, 128)


def _thresh_kernel(topk_ref, m_ref, t_ref):
    keys = lax.bitcast_convert_type(m_ref[...], jnp.int32)
    target = jnp.minimum(topk_ref[0], jnp.int32(MAXK))

    def body(_, lohi):
        lo, hi = lohi
        mid = lo + (hi - lo) // 2
        c = jnp.sum((keys >= mid).astype(jnp.int32))
        ok = c >= target
        return jnp.where(ok, mid, lo), jnp.where(ok, hi, mid)

    lo, _ = lax.fori_loop(0, 31, body, (jnp.int32(0), IMAX))
    mmax = jnp.max(keys) + 1
    for i in range(L):
        t_ref[i] = jnp.where(i == 1, target, jnp.where(i == 2, mmax, lo))


def _lane_iota():
    return lax.iota(jnp.int32, L)


def _sc_body(grad_ref, mx_ref, tk_ref, anew_ref, out_ref,
             mxv, idbuf, blk, lraw, lidx, keys, kidx, tkv, stat,
             sidx, sidxt, sraw, av128, sval,
             sp_raw, sp_idx, cnt_smem, lcnt_smem, aux_smem,
             sem_blk, sem_edge):
    sid = lax.axis_index("s")
    cid = lax.axis_index("c")
    lanes = _lane_iota()
    zi = jnp.zeros((L,), jnp.int32)

    @pl.when(sid == 0)
    def _():
        cnt_smem[0] = 0
    for q in range(SLICE // L):
        keys[pl.ds(q * L, L)] = zi
    zoff = pl.multiple_of(sid * SLICE, 8)
    pltpu.sync_copy(keys.at[pl.ds(0, SLICE)], sp_raw.at[pl.ds(zoff, SLICE)])
    pltpu.sync_copy(keys.at[pl.ds(0, SLICE)], sp_idx.at[pl.ds(zoff, SLICE)])
    for q in range(CAPB // L):
        idbuf[pl.ds(q * L, L)] = zi
    plsc.subcore_barrier()

    pltpu.sync_copy(mx_ref.at[pl.ds(pl.multiple_of(sid * BPT, 8), BPT)], mxv)
    pltpu.sync_copy(tk_ref, tkv)
    tk16 = tkv[...]
    t_key = jnp.sum(jnp.where(lanes == 0, tk16, 0))
    k_target = jnp.sum(jnp.where(lanes == 1, tk16, 0))
    hi_key = jnp.sum(jnp.where(lanes == 2, tk16, 0))

    def scan_body(g, cnt):
        m16 = mxv[pl.ds(pl.multiple_of(g * L, 8), L)]
        kk = lax.bitcast_convert_type(m16, jnp.int32)
        m = kk >= t_key
        ids = sid * BPT + g * L + lanes
        mi = m.astype(jnp.int32)
        wpos = cnt + plsc.cumsum(mi) - mi
        plsc.store_scatter(idbuf, [jnp.minimum(wpos, CAPB - 1)], ids, mask=m)
        return cnt + jnp.sum(mi)

    cnt = lax.fori_loop(0, BPT // L, scan_body, jnp.int32(0))
    cnt = jnp.minimum(cnt, jnp.int32(CAPB))

    lcnt_smem[0] = 0

    def chunk_body(c, _):
        @pl.when(c * L < cnt)
        def _():
            bidv = idbuf[pl.ds(pl.multiple_of(c * L, 8), L)]
            for r in range(L):
                @pl.when(c * L + r < cnt)
                def _(r=r):
                    bid = jnp.sum(jnp.where(lanes == r, bidv, 0))
                    grow = bid >> 5
                    gcol = (bid & (BPR - 1)) * BLK
                    pltpu.async_copy(grad_ref.at[grow, pl.ds(gcol, BLK)],
                                     blk.at[r], sem_blk)
            for r in range(L):
                @pl.when(c * L + r < cnt)
                def _(r=r):
                    bid = jnp.sum(jnp.where(lanes == r, bidv, 0))
                    grow = bid >> 5
                    gcol = (bid & (BPR - 1)) * BLK
                    pltpu.make_async_copy(grad_ref.at[grow, pl.ds(gcol, BLK)],
                                          blk.at[r], sem_blk).wait()
            for r in range(L):
                @pl.when(c * L + r < cnt)
                def _(r=r):
                    bid = jnp.sum(jnp.where(lanes == r, bidv, 0))

                    def grp_body(g, lc):
                        raw = blk[r, pl.ds(pl.multiple_of(g * L, 8), L)]
                        bits = lax.bitcast_convert_type(raw, jnp.int32)
                        key = bits & 0x7FFFFFFF
                        pos = bid * BLK + g * L + lanes
                        row = pos >> 12
                        col = pos & (N - 1)
                        sel = (col > row) & (key >= t_key)
                        si = sel.astype(jnp.int32)
                        wpos = jnp.minimum(lc + plsc.cumsum(si) - si,
                                           CAPC - 1)
                        plsc.store_scatter(lraw, [wpos], bits, mask=sel)
                        plsc.store_scatter(lidx, [wpos], pos, mask=sel)
                        return lc + jnp.sum(si)

                    lc = lax.fori_loop(0, BLK // L, grp_body, lcnt_smem[0])
                    lcnt_smem[0] = jnp.minimum(lc, jnp.int32(CAPC))
        return 0

    lax.fori_loop(0, CAPB // L, chunk_body, 0)
    lcnt = lcnt_smem[0]

    lpad = (lcnt + (L - 1)) & ~(L - 1)

    @pl.when(lpad > lcnt)
    def _():
        toff = pl.multiple_of(lpad - L, 8)
        tail = lraw[pl.ds(toff, L)]
        m = (lpad - L + lanes) < lcnt
        lraw[pl.ds(toff, L)] = jnp.where(m, tail, 0)
        tidx = lidx[pl.ds(toff, L)]
        lidx[pl.ds(toff, L)] = jnp.where(m, tidx, 0)

    off = plsc.fetch_and_add(cnt_smem.at[0], lpad, subcore_id=0)
    off = jnp.minimum(off, jnp.int32(SPC - CAPC))

    def pack_body(q, _):
        @pl.when(q * L < lpad)
        def _():
            poff = pl.multiple_of(off + q * L, 8)
            pltpu.sync_copy(lraw.at[pl.ds(q * L, L)],
                            sp_raw.at[pl.ds(poff, L)])
            pltpu.sync_copy(lidx.at[pl.ds(q * L, L)],
                            sp_idx.at[pl.ds(poff, L)])
        return 0

    lax.fori_loop(0, CAPC // L, pack_body, 0)
    plsc.subcore_barrier()

    pltpu.sync_copy(sp_raw, keys)
    total = plsc.fetch_and_add(cnt_smem.at[0], 0, subcore_id=0)
    ngrp = jnp.minimum((total + (L - 1)) >> 4, jnp.int32(SPC // L))

    def count_ge(x):
        def cbody(q, acc):
            kk = keys[pl.ds(pl.multiple_of(q * L, 8), L)] & 0x7FFFFFFF
            return acc + jnp.sum((kk >= x).astype(jnp.int32))
        return lax.fori_loop(0, ngrp, cbody, jnp.int32(0))

    def bis_body(_, lohi):
        lo, hi = lohi
        mid = lo + (hi - lo) // 2
        ok = count_ge(mid) >= k_target
        return jnp.where(ok, mid, lo), jnp.where(ok, hi, mid)

    tfin, _ = lax.fori_loop(0, 31, bis_body, (t_key, hi_key))

    pltpu.sync_copy(sp_idx, kidx)
    base = sid * SLICE

    aux_smem[1] = -1

    for gq in range(SLICE // L):
        goff = pl.multiple_of(base + gq * L, 8)
        raw = keys[pl.ds(goff, L)]
        idx = kidx[pl.ds(goff, L)]
        key = raw & 0x7FFFFFFF
        sel = (key >= tfin) & ((idx & 1) == cid)
        nsel = jnp.sum(sel.astype(jnp.int32))

        @pl.when(nsel > 0)
        def _(raw=raw, idx=idx, sel=sel, gq=gq):
            fl = plsc.all_reduce_ffs(sel)
            first_idx = jnp.sum(jnp.where(lanes == fl, idx, 0))
            raw_first = jnp.sum(jnp.where(lanes == fl, raw, 0))

            @pl.when(aux_smem[1] < 0)
            def _():
                aux_smem[1] = first_idx
                aux_smem[2] = raw_first
            sidx[pl.ds(gq * L, L)] = jnp.where(sel, idx, first_idx)
            sraw[pl.ds(gq * L, L)] = jnp.where(sel, raw, raw_first)

        @pl.when(nsel == 0)
        def _(gq=gq):
            sidx[pl.ds(gq * L, L)] = jnp.full((L,), -1, jnp.int32)

    @pl.when(aux_smem[1] >= 0)
    def _():
        ffi = aux_smem[1]
        ffr = aux_smem[2]
        for gq in range(SLICE // L):
            v = sidx[pl.ds(gq * L, L)]
            m = v < 0
            sidx[pl.ds(gq * L, L)] = jnp.where(m, ffi, v)
            w = sraw[pl.ds(gq * L, L)]
            sraw[pl.ds(gq * L, L)] = jnp.where(m, ffr, w)
        pltpu.async_copy(anew_ref.at[sidx], av128, sem_edge).wait()
        for gq in range(SLICE // L):
            av = av128[pl.ds(gq * L, L)]
            gv = lax.bitcast_convert_type(sraw[pl.ds(gq * L, L)], jnp.float32)
            exist = av > 0.0
            dec = exist & (gv <= 0.0)
            inc = (~exist) & (gv >= 0.0)
            d = jnp.where(dec, jnp.float32(-STEP),
                          jnp.where(inc, jnp.float32(STEP), jnp.float32(0.0)))
            sval[pl.ds(gq * L, L)] = av + d
            iv = sidx[pl.ds(gq * L, L)]
            u = iv >> 12
            v = iv & (N - 1)
            sidxt[pl.ds(gq * L, L)] = (v << 12) | u
        pltpu.async_copy(sval, anew_ref.at[sidx], sem_edge).wait()
        pltpu.async_copy(sval, anew_ref.at[sidxt], sem_edge).wait()

    @pl.when((sid == 0) & (cid == 0))
    def _():
        stat[pl.ds(0, L)] = jnp.full((L,), tfin, jnp.int32)
        pltpu.sync_copy(stat.at[pl.ds(0, L)], out_ref)


@jax.jit
def _impl(gradA, A_logits, topk):
    maxes, a_new = pl.pallas_call(
        _maxes_copy_kernel,
        grid=(NBANDS,),
        in_specs=[
            pl.BlockSpec((BAND, N), lambda b: (b, 0)),
            pl.BlockSpec((BAND, N), lambda b: (b, 0)),
        ],
        out_specs=[
            pl.BlockSpec((BAND * BPR // 128, 128), lambda b: (b, 0)),
            pl.BlockSpec((BAND * N // 128, 128), lambda b: (b, 0)),
        ],
        out_shape=[
            jax.ShapeDtypeStruct((NBLK // 128, 128), jnp.float32),
            jax.ShapeDtypeStruct((NSQ // 128, 128), jnp.float32),
        ],
    )(gradA, A_logits)

    topk_arr = jnp.asarray(topk, jnp.int32).reshape((1,))
    tk = pl.pallas_call(
        _thresh_kernel,
        in_specs=[
            pl.BlockSpec(memory_space=pltpu.SMEM),
            pl.BlockSpec((NBLK // 128, 128), lambda: (0, 0)),
        ],
        out_specs=pl.BlockSpec(memory_space=pltpu.SMEM),
        out_shape=jax.ShapeDtypeStruct((L,), jnp.int32),
    )(topk_arr, maxes)

    sc = pl.kernel(
        _sc_body,
        out_type=jax.ShapeDtypeStruct((L,), jnp.int32),
        mesh=plsc.VectorSubcoreMesh(
            core_axis_name="c", subcore_axis_name="s",
            num_cores=NCORE, num_subcores=NSUB),
        compiler_params=pltpu.CompilerParams(needs_layout_passes=False),
        scratch_types=[
            pltpu.VMEM((BPT,), jnp.float32),
            pltpu.VMEM((CAPB,), jnp.int32),
            pltpu.VMEM((L, BLK), jnp.float32),
            pltpu.VMEM((CAPC,), jnp.int32),
            pltpu.VMEM((CAPC,), jnp.int32),
            pltpu.VMEM((SPC,), jnp.int32),
            pltpu.VMEM((SPC,), jnp.int32),
            pltpu.VMEM((L,), jnp.int32),
            pltpu.VMEM((L,), jnp.int32),
            pltpu.VMEM((SLICE,), jnp.int32),
            pltpu.VMEM((SLICE,), jnp.int32),
            pltpu.VMEM((SLICE,), jnp.int32),
            pltpu.VMEM((SLICE,), jnp.float32),
            pltpu.VMEM((SLICE,), jnp.float32),
            pltpu.VMEM_SHARED((SPC,), jnp.int32),
            pltpu.VMEM_SHARED((SPC,), jnp.int32),
            pltpu.SMEM((1,), jnp.int32),
            pltpu.SMEM((1,), jnp.int32),
            pltpu.SMEM((4,), jnp.int32),
            pltpu.SemaphoreType.DMA,
            pltpu.SemaphoreType.DMA,
        ],
    )
    mx_flat = maxes.reshape(NBLK)
    anew_ref = jax.new_ref(a_new.reshape(NSQ))
    _ = sc(gradA, mx_flat, tk, anew_ref)
    return anew_ref[...].reshape(N, N)


def kernel(gradA, A_logits, topk):
    return _impl(gradA, A_logits, topk)


# speedup vs baseline: 1.9978x; 1.0067x over previous
import functools

import jax
import jax.numpy as jnp
import numpy as np
from jax import lax
from jax.experimental import pallas as pl
from jax.experimental.pallas import tpu as pltpu
from jax.experimental.pallas import tpu_sc as plsc

N = 4096
NSQ = N * N
BAND = 512
NBANDS = N // BAND
BLK = 128
NBLK = NSQ // BLK
BPR = N // BLK
STEP = 2.5
MAXK = 1024

NCORE = 2
NSUB = 16
L = 16
BPT = NBLK // NSUB
CAPB = 256
CAPC = 256
SPC = 2048
SLICE = SPC // NSUB
IMIN = np.int32(-2147483648)
IMAX = np.int32(2147483647)


MROWS = BAND * BPR // 128


def _maxes_copy_kernel(topk_ref, g_ref, a_ref, m_ref, out_ref, t_ref,
                       macc_ref):
    b = pl.program_id(0)
    g = g_ref[...]
    rows = lax.broadcasted_iota(jnp.int32, (BAND, N), 0) + b * BAND
    cols = lax.broadcasted_iota(jnp.int32, (BAND, N), 1)
    s = jnp.where(cols > rows, jnp.abs(g), 0.0)
    m = jnp.max(s.reshape(BAND, BPR, BLK), axis=-1)
    m2 = m.reshape(MROWS, 128)
    m_ref[...] = m2
    macc_ref[pl.ds(b * MROWS, MROWS), :] = m2
    a = jnp.where(cols == rows, jnp.float32(-10.0), a_ref[...])
    out_ref[...] = a.reshape(BAND * N // 128, 128)

    @pl.when(b == NBANDS - 1)
    def _():
        keys = lax.bitcast_convert_type(macc_ref[...], jnp.int32)
        target = jnp.minimum(topk_ref[0], jnp.int32(MAXK))

        def body(_, lohi):
            lo, hi = lohi
            mid = lo + (hi - lo) // 2
            c = jnp.sum((keys >= mid).astype(jnp.int32))
            ok = c >= target
            return jnp.where(ok, mid, lo), jnp.where(ok, hi, mid)

        lo, _ = lax.fori_loop(0, 31, body, (jnp.int32(0), IMAX))
        mmax = jnp.max(keys) + 1
        for i in range(L):
            t_ref[i] = jnp.where(i == 1, target,
                                 jnp.where(i == 2, mmax, lo))


def _lane_iota():
    return lax.iota(jnp.int32, L)


def _sc_body(grad_ref, mx_ref, tk_ref, anew_ref, out_ref,
             mxv, idbuf, blk, lraw, lidx, keys, kidx, tkv, stat,
             sidx, sidxt, sraw, av128, sval,
             sp_raw, sp_idx, cnt_smem, lcnt_smem, aux_smem,
             sem_blk, sem_edge):
    sid = lax.axis_index("s")
    cid = lax.axis_index("c")
    lanes = _lane_iota()
    zi = jnp.zeros((L,), jnp.int32)

    @pl.when(sid == 0)
    def _():
        cnt_smem[0] = 0
    for q in range(SLICE // L):
        keys[pl.ds(q * L, L)] = zi
    zoff = pl.multiple_of(sid * SLICE, 8)
    pltpu.sync_copy(keys.at[pl.ds(0, SLICE)], sp_raw.at[pl.ds(zoff, SLICE)])
    pltpu.sync_copy(keys.at[pl.ds(0, SLICE)], sp_idx.at[pl.ds(zoff, SLICE)])
    for q in range(CAPB // L):
        idbuf[pl.ds(q * L, L)] = zi
    plsc.subcore_barrier()

    pltpu.sync_copy(mx_ref.at[pl.ds(pl.multiple_of(sid * BPT, 8), BPT)], mxv)
    pltpu.sync_copy(tk_ref, tkv)
    tk16 = tkv[...]
    t_key = jnp.sum(jnp.where(lanes == 0, tk16, 0))
    k_target = jnp.sum(jnp.where(lanes == 1, tk16, 0))
    hi_key = jnp.sum(jnp.where(lanes == 2, tk16, 0))

    def scan_body(g, cnt):
        m16 = mxv[pl.ds(pl.multiple_of(g * L, 8), L)]
        kk = lax.bitcast_convert_type(m16, jnp.int32)
        m = kk >= t_key
        ids = sid * BPT + g * L + lanes
        mi = m.astype(jnp.int32)
        wpos = cnt + plsc.cumsum(mi) - mi
        plsc.store_scatter(idbuf, [jnp.minimum(wpos, CAPB - 1)], ids, mask=m)
        return cnt + jnp.sum(mi)

    cnt = lax.fori_loop(0, BPT // L, scan_body, jnp.int32(0))
    cnt = jnp.minimum(cnt, jnp.int32(CAPB))

    lcnt_smem[0] = 0

    def chunk_body(c, _):
        @pl.when(c * L < cnt)
        def _():
            bidv = idbuf[pl.ds(pl.multiple_of(c * L, 8), L)]
            for r in range(L):
                @pl.when(c * L + r < cnt)
                def _(r=r):
                    bid = jnp.sum(jnp.where(lanes == r, bidv, 0))
                    grow = bid >> 5
                    gcol = (bid & (BPR - 1)) * BLK
                    pltpu.async_copy(grad_ref.at[grow, pl.ds(gcol, BLK)],
                                     blk.at[r], sem_blk)
            for r in range(L):
                @pl.when(c * L + r < cnt)
                def _(r=r):
                    bid = jnp.sum(jnp.where(lanes == r, bidv, 0))
                    grow = bid >> 5
                    gcol = (bid & (BPR - 1)) * BLK
                    pltpu.make_async_copy(grad_ref.at[grow, pl.ds(gcol, BLK)],
                                          blk.at[r], sem_blk).wait()
            for r in range(L):
                @pl.when(c * L + r < cnt)
                def _(r=r):
                    bid = jnp.sum(jnp.where(lanes == r, bidv, 0))

                    def grp_body(g, lc):
                        raw = blk[r, pl.ds(pl.multiple_of(g * L, 8), L)]
                        bits = lax.bitcast_convert_type(raw, jnp.int32)
                        key = bits & 0x7FFFFFFF
                        pos = bid * BLK + g * L + lanes
                        row = pos >> 12
                        col = pos & (N - 1)
                        sel = (col > row) & (key >= t_key)
                        si = sel.astype(jnp.int32)
                        wpos = jnp.minimum(lc + plsc.cumsum(si) - si,
                                           CAPC - 1)
                        plsc.store_scatter(lraw, [wpos], bits, mask=sel)
                        plsc.store_scatter(lidx, [wpos], pos, mask=sel)
                        return lc + jnp.sum(si)

                    lc = lax.fori_loop(0, BLK // L, grp_body, lcnt_smem[0])
                    lcnt_smem[0] = jnp.minimum(lc, jnp.int32(CAPC))
        return 0

    lax.fori_loop(0, CAPB // L, chunk_body, 0)
    lcnt = lcnt_smem[0]

    lpad = (lcnt + (L - 1)) & ~(L - 1)

    @pl.when(lpad > lcnt)
    def _():
        toff = pl.multiple_of(lpad - L, 8)
        tail = lraw[pl.ds(toff, L)]
        m = (lpad - L + lanes) < lcnt
        lraw[pl.ds(toff, L)] = jnp.where(m, tail, 0)
        tidx = lidx[pl.ds(toff, L)]
        lidx[pl.ds(toff, L)] = jnp.where(m, tidx, 0)

    off = plsc.fetch_and_add(cnt_smem.at[0], lpad, subcore_id=0)
    off = jnp.minimum(off, jnp.int32(SPC - CAPC))

    def pack_body(q, _):
        @pl.when(q * L < lpad)
        def _():
            poff = pl.multiple_of(off + q * L, 8)
            pltpu.sync_copy(lraw.at[pl.ds(q * L, L)],
                            sp_raw.at[pl.ds(poff, L)])
            pltpu.sync_copy(lidx.at[pl.ds(q * L, L)],
                            sp_idx.at[pl.ds(poff, L)])
        return 0

    lax.fori_loop(0, CAPC // L, pack_body, 0)
    plsc.subcore_barrier()

    pltpu.sync_copy(sp_raw, keys)
    total = plsc.fetch_and_add(cnt_smem.at[0], 0, subcore_id=0)
    ngrp = jnp.minimum((total + (L - 1)) >> 4, jnp.int32(SPC // L))

    def count_ge(x):
        def cbody(q, acc):
            kk = keys[pl.ds(pl.multiple_of(q * L, 8), L)] & 0x7FFFFFFF
            return acc + jnp.sum((kk >= x).astype(jnp.int32))
        return lax.fori_loop(0, ngrp, cbody, jnp.int32(0))

    def bis_body(_, lohi):
        lo, hi = lohi
        mid = lo + (hi - lo) // 2
        ok = count_ge(mid) >= k_target
        return jnp.where(ok, mid, lo), jnp.where(ok, hi, mid)

    tfin, _ = lax.fori_loop(0, 31, bis_body, (t_key, hi_key))

    pltpu.sync_copy(sp_idx, kidx)
    base = sid * SLICE

    aux_smem[1] = -1

    for gq in range(SLICE // L):
        goff = pl.multiple_of(base + gq * L, 8)
        raw = keys[pl.ds(goff, L)]
        idx = kidx[pl.ds(goff, L)]
        key = raw & 0x7FFFFFFF
        sel = (key >= tfin) & ((idx & 1) == cid)
        nsel = jnp.sum(sel.astype(jnp.int32))

        @pl.when(nsel > 0)
        def _(raw=raw, idx=idx, sel=sel, gq=gq):
            fl = plsc.all_reduce_ffs(sel)
            first_idx = jnp.sum(jnp.where(lanes == fl, idx, 0))
            raw_first = jnp.sum(jnp.where(lanes == fl, raw, 0))

            @pl.when(aux_smem[1] < 0)
            def _():
                aux_smem[1] = first_idx
                aux_smem[2] = raw_first
            sidx[pl.ds(gq * L, L)] = jnp.where(sel, idx, first_idx)
            sraw[pl.ds(gq * L, L)] = jnp.where(sel, raw, raw_first)

        @pl.when(nsel == 0)
        def _(gq=gq):
            sidx[pl.ds(gq * L, L)] = jnp.full((L,), -1, jnp.int32)

    @pl.when(aux_smem[1] >= 0)
    def _():
        ffi = aux_smem[1]
        ffr = aux_smem[2]
        for gq in range(SLICE // L):
            v = sidx[pl.ds(gq * L, L)]
            m = v < 0
            sidx[pl.ds(gq * L, L)] = jnp.where(m, ffi, v)
            w = sraw[pl.ds(gq * L, L)]
            sraw[pl.ds(gq * L, L)] = jnp.where(m, ffr, w)
        pltpu.async_copy(anew_ref.at[sidx], av128, sem_edge).wait()
        for gq in range(SLICE // L):
            av = av128[pl.ds(gq * L, L)]
            gv = lax.bitcast_convert_type(sraw[pl.ds(gq * L, L)], jnp.float32)
            exist = av > 0.0
            dec = exist & (gv <= 0.0)
            inc = (~exist) & (gv >= 0.0)
            d = jnp.where(dec, jnp.float32(-STEP),
                          jnp.where(inc, jnp.float32(STEP), jnp.float32(0.0)))
            sval[pl.ds(gq * L, L)] = av + d
            iv = sidx[pl.ds(gq * L, L)]
            u = iv >> 12
            v = iv & (N - 1)
            sidxt[pl.ds(gq * L, L)] = (v << 12) | u
        pltpu.async_copy(sval, anew_ref.at[sidx], sem_edge).wait()
        pltpu.async_copy(sval, anew_ref.at[sidxt], sem_edge).wait()

    @pl.when((sid == 0) & (cid == 0))
    def _():
        stat[pl.ds(0, L)] = jnp.full((L,), tfin, jnp.int32)
        pltpu.sync_copy(stat.at[pl.ds(0, L)], out_ref)


@jax.jit
def _impl(gradA, A_logits, topk):
    topk_arr = jnp.asarray(topk, jnp.int32).reshape((1,))
    maxes, a_new, tk = pl.pallas_call(
        _maxes_copy_kernel,
        grid=(NBANDS,),
        in_specs=[
            pl.BlockSpec(memory_space=pltpu.SMEM),
            pl.BlockSpec((BAND, N), lambda b: (b, 0)),
            pl.BlockSpec((BAND, N), lambda b: (b, 0)),
        ],
        out_specs=[
            pl.BlockSpec((BAND * BPR // 128, 128), lambda b: (b, 0)),
            pl.BlockSpec((BAND * N // 128, 128), lambda b: (b, 0)),
            pl.BlockSpec(memory_space=pltpu.SMEM),
        ],
        out_shape=[
            jax.ShapeDtypeStruct((NBLK // 128, 128), jnp.float32),
            jax.ShapeDtypeStruct((NSQ // 128, 128), jnp.float32),
            jax.ShapeDtypeStruct((L,), jnp.int32),
        ],
        scratch_shapes=[pltpu.VMEM((NBLK // 128, 128), jnp.float32)],
    )(topk_arr, gradA, A_logits)

    sc = pl.kernel(
        _sc_body,
        out_type=jax.ShapeDtypeStruct((L,), jnp.int32),
        mesh=plsc.VectorSubcoreMesh(
            core_axis_name="c", subcore_axis_name="s",
            num_cores=NCORE, num_subcores=NSUB),
        compiler_params=pltpu.CompilerParams(needs_layout_passes=False),
        scratch_types=[
            pltpu.VMEM((BPT,), jnp.float32),
            pltpu.VMEM((CAPB,), jnp.int32),
            pltpu.VMEM((L, BLK), jnp.float32),
            pltpu.VMEM((CAPC,), jnp.int32),
            pltpu.VMEM((CAPC,), jnp.int32),
            pltpu.VMEM((SPC,), jnp.int32),
            pltpu.VMEM((SPC,), jnp.int32),
            pltpu.VMEM((L,), jnp.int32),
            pltpu.VMEM((L,), jnp.int32),
            pltpu.VMEM((SLICE,), jnp.int32),
            pltpu.VMEM((SLICE,), jnp.int32),
            pltpu.VMEM((SLICE,), jnp.int32),
            pltpu.VMEM((SLICE,), jnp.float32),
            pltpu.VMEM((SLICE,), jnp.float32),
            pltpu.VMEM_SHARED((SPC,), jnp.int32),
            pltpu.VMEM_SHARED((SPC,), jnp.int32),
            pltpu.SMEM((1,), jnp.int32),
            pltpu.SMEM((1,), jnp.int32),
            pltpu.SMEM((4,), jnp.int32),
            pltpu.SemaphoreType.DMA,
            pltpu.SemaphoreType.DMA,
        ],
    )
    mx_flat = maxes.reshape(NBLK)
    anew_ref = jax.new_ref(a_new.reshape(NSQ))
    _ = sc(gradA, mx_flat, tk, anew_ref)
    return anew_ref[...].reshape(N, N)


def kernel(gradA, A_logits, topk):
    return _impl(gradA, A_logits, topk)
